# baseline (speedup 1.0000x reference)
import functools

import jax
import jax.numpy as jnp
from jax import lax
from jax.experimental import pallas as pl
from jax.experimental.pallas import tpu as pltpu
from jax.experimental.pallas import tpu_sc as plsc

_N = 10000
_E = 160000
_D = 256
_W1P, _W2P, _W3P, _W4P = 176, 128, 64, 16
_W1A, _W1B = 96, 80
_W1S = 96

_NC, _NS = 2, 16
_NW = _NC * _NS
_LSZ = 128
_NCHUNK = 40
_EPT = _NCHUNK * _LSZ
_EPAD = _NW * _EPT
_ACC_ROWS = 10112

_PREC = lax.Precision.DEFAULT


def _agg_pipeline(y_hbm, acc_sh, src_v, dst_v, bufs, gsem, ssem, nchunk):
    for b in range(3):
        pltpu.async_copy(y_hbm.at[src_v.at[b]], bufs.at[b], gsem[b])

    @pl.loop(0, nchunk, step=4)
    def _pipe(j):
        for b in range(4):
            t = j + b + 3
            bp = (b + 3) % 4

            @pl.when(t < nchunk)
            def _():
                @pl.when(t >= 4)
                def _():
                    pltpu.make_async_copy(
                        bufs.at[bp], acc_sh.at[dst_v.at[t - 4]],
                        ssem[bp]).wait()

                pltpu.async_copy(y_hbm.at[src_v.at[t]], bufs.at[bp], gsem[bp])

            pltpu.make_async_copy(y_hbm.at[src_v.at[j + b]], bufs.at[b],
                                  gsem[b]).wait()
            pltpu.async_copy(bufs.at[b], acc_sh.at[dst_v.at[j + b]], ssem[b],
                             add=True)

    for b in range(4):
        pltpu.make_async_copy(bufs.at[b],
                              acc_sh.at[dst_v.at[nchunk - 4 + b]],
                              ssem[b]).wait()


@functools.lru_cache(maxsize=None)
def _sc_agg_factory(w):
    mesh = plsc.VectorSubcoreMesh(core_axis_name="c", subcore_axis_name="s",
                                  num_cores=_NC, num_subcores=_NS)
    rows_pt = _ACC_ROWS // _NS

    @functools.partial(
        pl.kernel,
        out_type=jax.ShapeDtypeStruct((_NC, _ACC_ROWS, w), jnp.float32),
        mesh=mesh,
        scratch_types=[
            pltpu.VMEM((_NCHUNK, _LSZ), jnp.int32),
            pltpu.VMEM((_NCHUNK, _LSZ), jnp.int32),
            pltpu.VMEM((4, _LSZ, w), jnp.float32),
            pltpu.VMEM_SHARED((_ACC_ROWS, w), jnp.float32),
        ] + [pltpu.SemaphoreType.DMA] * 8,
        compiler_params=pltpu.CompilerParams(use_tc_tiling_on_sc=False),
    )
    def agg(y_hbm, src_hbm, dst_hbm, zeros_hbm, p_hbm, src_v, dst_v, bufs,
            acc_sh, *sems):
        gsem, ssem = sems[:4], sems[4:]
        c = lax.axis_index("c")
        s = lax.axis_index("s")
        wid = c * _NS + s

        pltpu.async_copy(src_hbm.at[wid], src_v, gsem[0])
        pltpu.async_copy(dst_hbm.at[wid], dst_v, gsem[0])

        pltpu.sync_copy(zeros_hbm, bufs.at[0])
        zbase = s * rows_pt
        for k in range(4):
            pltpu.sync_copy(bufs.at[0],
                            acc_sh.at[pl.ds(zbase + k * _LSZ, _LSZ)])
        rem = rows_pt - 4 * _LSZ
        pltpu.sync_copy(bufs.at[0].at[pl.ds(0, rem)],
                        acc_sh.at[pl.ds(zbase + 4 * _LSZ, rem)])

        pltpu.make_async_copy(src_hbm.at[wid], src_v, gsem[0]).wait()
        pltpu.make_async_copy(dst_hbm.at[wid], dst_v, gsem[0]).wait()
        plsc.subcore_barrier()

        _agg_pipeline(y_hbm, acc_sh, src_v, dst_v, bufs, gsem, ssem, _NCHUNK)

        plsc.subcore_barrier()
        pltpu.sync_copy(acc_sh.at[pl.ds(zbase, rows_pt)],
                        p_hbm.at[c, pl.ds(zbase, rows_pt)])

    return agg


_EPT2 = _EPAD // _NS


@functools.lru_cache(maxsize=None)
def _sc_split_factory(w, lsz, nchunk):
    mesh = plsc.VectorSubcoreMesh(core_axis_name="c", subcore_axis_name="s",
                                  num_cores=_NC, num_subcores=_NS)
    rows_pt = _ACC_ROWS // _NS
    nz, zrem = divmod(rows_pt, lsz)

    @functools.partial(
        pl.kernel,
        out_type=jax.ShapeDtypeStruct((_NC, _ACC_ROWS, w), jnp.float32),
        mesh=mesh,
        scratch_types=[
            pltpu.VMEM((nchunk, lsz), jnp.int32),
            pltpu.VMEM((nchunk, lsz), jnp.int32),
            pltpu.VMEM((4, lsz, w), jnp.float32),
            pltpu.VMEM_SHARED((_ACC_ROWS, w), jnp.float32),
        ] + [pltpu.SemaphoreType.DMA] * 8,
        compiler_params=pltpu.CompilerParams(use_tc_tiling_on_sc=False),
    )
    def agg(y_hbm, src_hbm, dst_hbm, zeros_hbm, p_hbm, src_v, dst_v, bufs,
            acc_sh, *sems):
        gsem, ssem = sems[:4], sems[4:]
        c = lax.axis_index("c")
        s = lax.axis_index("s")

        pltpu.async_copy(src_hbm.at[c, s], src_v, gsem[0])
        pltpu.async_copy(dst_hbm.at[s], dst_v, gsem[0])

        pltpu.sync_copy(zeros_hbm, bufs.at[0])
        zbase = s * rows_pt
        for k in range(nz):
            pltpu.sync_copy(bufs.at[0],
                            acc_sh.at[pl.ds(zbase + k * lsz, lsz)])
        if zrem:
            pltpu.sync_copy(bufs.at[0].at[pl.ds(0, zrem)],
                            acc_sh.at[pl.ds(zbase + nz * lsz, zrem)])

        pltpu.make_async_copy(src_hbm.at[c, s], src_v, gsem[0]).wait()
        pltpu.make_async_copy(dst_hbm.at[s], dst_v, gsem[0]).wait()
        plsc.subcore_barrier()

        _agg_pipeline(y_hbm, acc_sh, src_v, dst_v, bufs, gsem, ssem, nchunk)

        plsc.subcore_barrier()
        pltpu.sync_copy(acc_sh.at[pl.ds(zbase, rows_pt)],
                        p_hbm.at[c, pl.ds(zbase, rows_pt)])

    return agg


_R = 400
_GRID = _N // _R


def _dot(a, b):
    return jnp.dot(a, b, preferred_element_type=jnp.float32, precision=_PREC)


def _tcy1_body(x_ref, wl_ref, ys_ref):
    y = _dot(x_ref[...], wl_ref[...])
    ya = y[:, :_W1A]
    yb = y[:, _W1A:]
    col = lax.broadcasted_iota(jnp.int32, (_R, _W1B), 1)
    yb = jnp.where(col == _W1B - 1, 1.0, yb)
    ys_ref[0] = ya
    ys_ref[1] = jnp.concatenate(
        [yb, jnp.zeros((_R, _W1S - _W1B), jnp.float32)], axis=1)


def _tcy1(x, wlT):
    return pl.pallas_call(
        _tcy1_body,
        grid=(_GRID,),
        in_specs=[
            pl.BlockSpec((_R, _D), lambda i: (i, 0)),
            pl.BlockSpec((_D, _W1P), lambda i: (0, 0)),
        ],
        out_specs=pl.BlockSpec((2, _R, _W1S), lambda i: (0, i, 0)),
        out_shape=jax.ShapeDtypeStruct((2, _N, _W1S), jnp.float32),
    )(x, wlT)


def _tcz1_body(x_ref, wr_ref, b_ref, z_ref):
    z_ref[...] = _dot(x_ref[...], wr_ref[...]) + b_ref[...]


def _tcz1(x, wrT, b):
    return pl.pallas_call(
        _tcz1_body,
        grid=(_GRID,),
        in_specs=[
            pl.BlockSpec((_R, _D), lambda i: (i, 0)),
            pl.BlockSpec((_D, _W1P), lambda i: (0, 0)),
            pl.BlockSpec((1, _W1P), lambda i: (0, 0)),
        ],
        out_specs=pl.BlockSpec((_R, _W1P), lambda i: (i, 0)),
        out_shape=jax.ShapeDtypeStruct((_N, _W1P), jnp.float32),
    )(x, wrT, b)


def _h2(pa_ref, pb_ref, z_ref):
    pa = pa_ref[...]
    pb = pb_ref[...]
    cnt = pb[:, _W1B - 1:_W1B]
    rinv = 1.0 / jnp.maximum(cnt, 1.0)
    h = jnp.concatenate([pa, pb[:, :_W1B]], axis=1) * rinv + z_ref[...]
    return h, rinv


def _tc2_body(pa_ref, pb_ref, z_ref, wl_ref, wr_ref, b_ref,
              ys_ref, z2_ref, rinv_ref):
    h, rinv = _h2(pa_ref, pb_ref, z_ref)
    y = _dot(h, wl_ref[...])
    hw = _W2P // 2
    ys_ref[0] = y[:, :hw]
    ys_ref[1] = y[:, hw:]
    z2_ref[...] = _dot(h, wr_ref[...]) + b_ref[...]
    rinv_ref[...] = rinv


def _tc2(pa, pb, z, wlT, wrT, b):
    return pl.pallas_call(
        _tc2_body,
        grid=(_GRID,),
        in_specs=[
            pl.BlockSpec((_R, _W1S), lambda i: (i, 0)),
            pl.BlockSpec((_R, _W1S), lambda i: (i, 0)),
            pl.BlockSpec((_R, _W1P), lambda i: (i, 0)),
            pl.BlockSpec((_W1P, _W2P), lambda i: (0, 0)),
            pl.BlockSpec((_W1P, _W2P), lambda i: (0, 0)),
            pl.BlockSpec((1, _W2P), lambda i: (0, 0)),
        ],
        out_specs=[
            pl.BlockSpec((2, _R, _W2P // 2), lambda i: (0, i, 0)),
            pl.BlockSpec((_R, _W2P), lambda i: (i, 0)),
            pl.BlockSpec((_R, 1), lambda i: (i, 0)),
        ],
        out_shape=[
            jax.ShapeDtypeStruct((2, _N, _W2P // 2), jnp.float32),
            jax.ShapeDtypeStruct((_N, _W2P), jnp.float32),
            jax.ShapeDtypeStruct((_N, 1), jnp.float32),
        ],
    )(pa, pb, z, wlT, wrT, b)


def _tcmid_s2s_body(pa_ref, pb_ref, z_ref, rinv_ref, wl_ref, wr_ref, b_ref,
                    ys_ref, z2_ref):
    h = (jnp.concatenate([pa_ref[...], pb_ref[...]], axis=1)
         * rinv_ref[...] + z_ref[...])
    y = _dot(h, wl_ref[...])
    hw = y.shape[1] // 2
    ys_ref[0] = y[:, :hw]
    ys_ref[1] = y[:, hw:]
    z2_ref[...] = _dot(h, wr_ref[...]) + b_ref[...]


def _tcmid_s2s(pa, pb, z, rinv, wlT, wrT, b, wi, wo):
    return pl.pallas_call(
        _tcmid_s2s_body,
        grid=(_GRID,),
        in_specs=[
            pl.BlockSpec((_R, wi // 2), lambda i: (i, 0)),
            pl.BlockSpec((_R, wi // 2), lambda i: (i, 0)),
            pl.BlockSpec((_R, wi), lambda i: (i, 0)),
            pl.BlockSpec((_R, 1), lambda i: (i, 0)),
            pl.BlockSpec((wi, wo), lambda i: (0, 0)),
            pl.BlockSpec((wi, wo), lambda i: (0, 0)),
            pl.BlockSpec((1, wo), lambda i: (0, 0)),
        ],
        out_specs=[
            pl.BlockSpec((2, _R, wo // 2), lambda i: (0, i, 0)),
            pl.BlockSpec((_R, wo), lambda i: (i, 0)),
        ],
        out_shape=[
            jax.ShapeDtypeStruct((2, _N, wo // 2), jnp.float32),
            jax.ShapeDtypeStruct((_N, wo), jnp.float32),
        ],
    )(pa, pb, z, rinv, wlT, wrT, b)


def _tcmid_s2f_body(pa_ref, pb_ref, z_ref, rinv_ref, wl_ref, wr_ref, b_ref,
                    y_ref, z2_ref):
    h = (jnp.concatenate([pa_ref[...], pb_ref[...]], axis=1)
         * rinv_ref[...] + z_ref[...])
    y_ref[...] = _dot(h, wl_ref[...])
    z2_ref[...] = _dot(h, wr_ref[...]) + b_ref[...]


def _tcmid_s2f(pa, pb, z, rinv, wlT, wrT, b, wi, wo):
    return pl.pallas_call(
        _tcmid_s2f_body,
        grid=(_GRID,),
        in_specs=[
            pl.BlockSpec((_R, wi // 2), lambda i: (i, 0)),
            pl.BlockSpec((_R, wi // 2), lambda i: (i, 0)),
            pl.BlockSpec((_R, wi), lambda i: (i, 0)),
            pl.BlockSpec((_R, 1), lambda i: (i, 0)),
            pl.BlockSpec((wi, wo), lambda i: (0, 0)),
            pl.BlockSpec((wi, wo), lambda i: (0, 0)),
            pl.BlockSpec((1, wo), lambda i: (0, 0)),
        ],
        out_specs=[
            pl.BlockSpec((_R, wo), lambda i: (i, 0)),
            pl.BlockSpec((_R, wo), lambda i: (i, 0)),
        ],
        out_shape=[jax.ShapeDtypeStruct((_N, wo), jnp.float32)] * 2,
    )(pa, pb, z, rinv, wlT, wrT, b)


def _tc5_body(p0_ref, p1_ref, z_ref, rinv_ref, out_ref):
    h = (p0_ref[...] + p1_ref[...]) * rinv_ref[...] + z_ref[...]
    out_ref[...] = h[:, 0:1]


def _tc5(p0, p1, z, rinv):
    return pl.pallas_call(
        _tc5_body,
        grid=(_GRID,),
        in_specs=[
            pl.BlockSpec((_R, _W4P), lambda i: (i, 0)),
            pl.BlockSpec((_R, _W4P), lambda i: (i, 0)),
            pl.BlockSpec((_R, _W4P), lambda i: (i, 0)),
            pl.BlockSpec((_R, 1), lambda i: (i, 0)),
        ],
        out_specs=pl.BlockSpec((_R, 1), lambda i: (i, 0)),
        out_shape=jax.ShapeDtypeStruct((_N, 1), jnp.float32),
    )(p0, p1, z, rinv)


def _padT(W, rows, cols):
    Wt = W.T
    return jnp.pad(Wt, ((0, rows - Wt.shape[0]), (0, cols - Wt.shape[1])))


def _padb(b, cols):
    return jnp.pad(b, (0, cols - b.shape[0])).reshape(1, cols)


def kernel(x, edge_index, W1l, b1l, W1r, W2l, b2l, W2r, W3l, b3l, W3r,
           W4l, b4l, W4r):
    f32 = jnp.float32
    src = edge_index[0]
    dst = edge_index[1]
    pad_idx = (jnp.arange(_EPAD - _E, dtype=jnp.int32) % 16)
    src_p = jnp.concatenate([src, pad_idx])
    dst_p = jnp.concatenate([dst, _N + pad_idx])
    srcr = src_p.reshape(_NW, _NCHUNK, _LSZ)
    dstr = dst_p.reshape(_NW, _NCHUNK, _LSZ)
    srcS = jnp.stack([src_p, src_p + _N])
    srcS1 = srcS.reshape(_NC, _NS, 160, 64)
    dstS1 = dst_p.reshape(_NS, 160, 64)
    srcS2 = srcS.reshape(_NC, _NS, 80, 128)
    dstS2 = dst_p.reshape(_NS, 80, 128)

    yS = _tcy1(x, _padT(W1l, _D, _W1P))
    p1 = _sc_split_factory(_W1S, 64, 160)(
        yS.reshape(2 * _N, _W1S), srcS1, dstS1, jnp.zeros((64, _W1S), f32))
    z1 = _tcz1(x, _padT(W1r, _D, _W1P), _padb(b1l, _W1P))
    y2s, z2, rinv = _tc2(p1[0], p1[1], z1, _padT(W2l, _W1P, _W2P),
                         _padT(W2r, _W1P, _W2P), _padb(b2l, _W2P))
    p2 = _sc_split_factory(_W2P // 2, 128, 80)(
        y2s.reshape(2 * _N, _W2P // 2), srcS2, dstS2,
        jnp.zeros((128, _W2P // 2), f32))
    y3s, z3 = _tcmid_s2s(p2[0], p2[1], z2, rinv, _padT(W3l, _W2P, _W3P),
                         _padT(W3r, _W2P, _W3P), _padb(b3l, _W3P),
                         _W2P, _W3P)
    p3 = _sc_split_factory(_W3P // 2, 128, 80)(
        y3s.reshape(2 * _N, _W3P // 2), srcS2, dstS2,
        jnp.zeros((128, _W3P // 2), f32))
    y4, z4 = _tcmid_s2f(p3[0], p3[1], z3, rinv, _padT(W4l, _W3P, _W4P),
                        _padT(W4r, _W3P, _W4P), _padb(b4l, _W4P),
                        _W3P, _W4P)
    p4 = _sc_agg_factory(_W4P)(y4, srcr, dstr, jnp.zeros((_LSZ, _W4P), f32))
    return _tc5(p4[0], p4[1], z4, rinv)


# speedup vs baseline: 1.0668x; 1.0668x over previous
import functools

import jax
import jax.numpy as jnp
from jax import lax
from jax.experimental import pallas as pl
from jax.experimental.pallas import tpu as pltpu
from jax.experimental.pallas import tpu_sc as plsc

_N = 10000
_E = 160000
_D = 256
_W1P, _W2P, _W3P, _W4P = 176, 128, 64, 16
_W1A, _W1B = 96, 80
_W1S = 96

_NC, _NS = 2, 16
_NW = _NC * _NS
_LSZ = 128
_NCHUNK = 40
_EPT = _NCHUNK * _LSZ
_EPAD = _NW * _EPT
_ACC_ROWS = 10112

_PREC = lax.Precision.DEFAULT


def _agg_pipeline(y_hbm, acc_sh, src_v, dst_v, bufs, gsem, ssem, nchunk):
    for b in range(3):
        pltpu.async_copy(y_hbm.at[src_v.at[b]], bufs.at[b], gsem[b])

    @pl.loop(0, nchunk, step=4)
    def _pipe(j):
        for b in range(4):
            t = j + b + 3
            bp = (b + 3) % 4

            @pl.when(t < nchunk)
            def _():
                @pl.when(t >= 4)
                def _():
                    pltpu.make_async_copy(
                        bufs.at[bp], acc_sh.at[dst_v.at[t - 4]],
                        ssem[bp]).wait()

                pltpu.async_copy(y_hbm.at[src_v.at[t]], bufs.at[bp], gsem[bp])

            pltpu.make_async_copy(y_hbm.at[src_v.at[j + b]], bufs.at[b],
                                  gsem[b]).wait()
            pltpu.async_copy(bufs.at[b], acc_sh.at[dst_v.at[j + b]], ssem[b],
                             add=True)

    for b in range(4):
        pltpu.make_async_copy(bufs.at[b],
                              acc_sh.at[dst_v.at[nchunk - 4 + b]],
                              ssem[b]).wait()


@functools.lru_cache(maxsize=None)
def _sc_agg_factory(w, lsz, nchunk):
    mesh = plsc.VectorSubcoreMesh(core_axis_name="c", subcore_axis_name="s",
                                  num_cores=_NC, num_subcores=_NS)
    rows_pt = _ACC_ROWS // _NS
    nz, zrem = divmod(rows_pt, lsz)

    @functools.partial(
        pl.kernel,
        out_type=jax.ShapeDtypeStruct((_NC, _ACC_ROWS, w), jnp.float32),
        mesh=mesh,
        scratch_types=[
            pltpu.VMEM((nchunk, lsz), jnp.int32),
            pltpu.VMEM((nchunk, lsz), jnp.int32),
            pltpu.VMEM((4, lsz, w), jnp.float32),
            pltpu.VMEM_SHARED((_ACC_ROWS, w), jnp.float32),
        ] + [pltpu.SemaphoreType.DMA] * 8,
        compiler_params=pltpu.CompilerParams(use_tc_tiling_on_sc=False),
    )
    def agg(y_hbm, src_hbm, dst_hbm, zeros_hbm, p_hbm, src_v, dst_v, bufs,
            acc_sh, *sems):
        gsem, ssem = sems[:4], sems[4:]
        c = lax.axis_index("c")
        s = lax.axis_index("s")
        wid = c * _NS + s

        pltpu.async_copy(src_hbm.at[wid], src_v, gsem[0])
        pltpu.async_copy(dst_hbm.at[wid], dst_v, gsem[0])

        pltpu.sync_copy(zeros_hbm, bufs.at[0])
        zbase = s * rows_pt
        for k in range(nz):
            pltpu.sync_copy(bufs.at[0],
                            acc_sh.at[pl.ds(zbase + k * lsz, lsz)])
        if zrem:
            pltpu.sync_copy(bufs.at[0].at[pl.ds(0, zrem)],
                            acc_sh.at[pl.ds(zbase + nz * lsz, zrem)])

        pltpu.make_async_copy(src_hbm.at[wid], src_v, gsem[0]).wait()
        pltpu.make_async_copy(dst_hbm.at[wid], dst_v, gsem[0]).wait()
        plsc.subcore_barrier()

        _agg_pipeline(y_hbm, acc_sh, src_v, dst_v, bufs, gsem, ssem, nchunk)

        plsc.subcore_barrier()
        pltpu.sync_copy(acc_sh.at[pl.ds(zbase, rows_pt)],
                        p_hbm.at[c, pl.ds(zbase, rows_pt)])

    return agg


_EPT2 = _EPAD // _NS


@functools.lru_cache(maxsize=None)
def _sc_split_factory(w, lsz, nchunk):
    mesh = plsc.VectorSubcoreMesh(core_axis_name="c", subcore_axis_name="s",
                                  num_cores=_NC, num_subcores=_NS)
    rows_pt = _ACC_ROWS // _NS
    nz, zrem = divmod(rows_pt, lsz)

    @functools.partial(
        pl.kernel,
        out_type=jax.ShapeDtypeStruct((_NC, _ACC_ROWS, w), jnp.float32),
        mesh=mesh,
        scratch_types=[
            pltpu.VMEM((nchunk, lsz), jnp.int32),
            pltpu.VMEM((nchunk, lsz), jnp.int32),
            pltpu.VMEM((4, lsz, w), jnp.float32),
            pltpu.VMEM_SHARED((_ACC_ROWS, w), jnp.float32),
        ] + [pltpu.SemaphoreType.DMA] * 8,
        compiler_params=pltpu.CompilerParams(use_tc_tiling_on_sc=False),
    )
    def agg(y_hbm, src_hbm, dst_hbm, zeros_hbm, p_hbm, src_v, dst_v, bufs,
            acc_sh, *sems):
        gsem, ssem = sems[:4], sems[4:]
        c = lax.axis_index("c")
        s = lax.axis_index("s")

        pltpu.async_copy(src_hbm.at[c, s], src_v, gsem[0])
        pltpu.async_copy(dst_hbm.at[s], dst_v, gsem[0])

        pltpu.sync_copy(zeros_hbm, bufs.at[0])
        zbase = s * rows_pt
        for k in range(nz):
            pltpu.sync_copy(bufs.at[0],
                            acc_sh.at[pl.ds(zbase + k * lsz, lsz)])
        if zrem:
            pltpu.sync_copy(bufs.at[0].at[pl.ds(0, zrem)],
                            acc_sh.at[pl.ds(zbase + nz * lsz, zrem)])

        pltpu.make_async_copy(src_hbm.at[c, s], src_v, gsem[0]).wait()
        pltpu.make_async_copy(dst_hbm.at[s], dst_v, gsem[0]).wait()
        plsc.subcore_barrier()

        _agg_pipeline(y_hbm, acc_sh, src_v, dst_v, bufs, gsem, ssem, nchunk)

        plsc.subcore_barrier()
        pltpu.sync_copy(acc_sh.at[pl.ds(zbase, rows_pt)],
                        p_hbm.at[c, pl.ds(zbase, rows_pt)])

    return agg


_R = 400
_GRID = _N // _R


def _dot(a, b):
    return jnp.dot(a, b, preferred_element_type=jnp.float32, precision=_PREC)


def _tcy1_body(x_ref, wl_ref, ys_ref):
    y = _dot(x_ref[...], wl_ref[...])
    ya = y[:, :_W1A]
    yb = y[:, _W1A:]
    col = lax.broadcasted_iota(jnp.int32, (_R, _W1B), 1)
    yb = jnp.where(col == _W1B - 1, 1.0, yb)
    ys_ref[0] = ya
    ys_ref[1] = jnp.concatenate(
        [yb, jnp.zeros((_R, _W1S - _W1B), jnp.float32)], axis=1)


def _tcy1(x, wlT):
    return pl.pallas_call(
        _tcy1_body,
        grid=(_GRID,),
        in_specs=[
            pl.BlockSpec((_R, _D), lambda i: (i, 0)),
            pl.BlockSpec((_D, _W1P), lambda i: (0, 0)),
        ],
        out_specs=pl.BlockSpec((2, _R, _W1S), lambda i: (0, i, 0)),
        out_shape=jax.ShapeDtypeStruct((2, _N, _W1S), jnp.float32),
    )(x, wlT)


def _tcz1_body(x_ref, wr_ref, b_ref, z_ref):
    z_ref[...] = _dot(x_ref[...], wr_ref[...]) + b_ref[...]


def _tcz1(x, wrT, b):
    return pl.pallas_call(
        _tcz1_body,
        grid=(_GRID,),
        in_specs=[
            pl.BlockSpec((_R, _D), lambda i: (i, 0)),
            pl.BlockSpec((_D, _W1P), lambda i: (0, 0)),
            pl.BlockSpec((1, _W1P), lambda i: (0, 0)),
        ],
        out_specs=pl.BlockSpec((_R, _W1P), lambda i: (i, 0)),
        out_shape=jax.ShapeDtypeStruct((_N, _W1P), jnp.float32),
    )(x, wrT, b)


def _h2(pa_ref, pb_ref, z_ref):
    pa = pa_ref[...]
    pb = pb_ref[...]
    cnt = pb[:, _W1B - 1:_W1B]
    rinv = 1.0 / jnp.maximum(cnt, 1.0)
    h = jnp.concatenate([pa, pb[:, :_W1B]], axis=1) * rinv + z_ref[...]
    return h, rinv


def _tc2_body(pa_ref, pb_ref, z_ref, wl_ref, wr_ref, b_ref,
              y_ref, z2_ref, rinv_ref):
    h, rinv = _h2(pa_ref, pb_ref, z_ref)
    y_ref[...] = _dot(h, wl_ref[...])
    z2_ref[...] = _dot(h, wr_ref[...]) + b_ref[...]
    rinv_ref[...] = rinv


def _tc2(pa, pb, z, wlT, wrT, b):
    return pl.pallas_call(
        _tc2_body,
        grid=(_GRID,),
        in_specs=[
            pl.BlockSpec((_R, _W1S), lambda i: (i, 0)),
            pl.BlockSpec((_R, _W1S), lambda i: (i, 0)),
            pl.BlockSpec((_R, _W1P), lambda i: (i, 0)),
            pl.BlockSpec((_W1P, _W2P), lambda i: (0, 0)),
            pl.BlockSpec((_W1P, _W2P), lambda i: (0, 0)),
            pl.BlockSpec((1, _W2P), lambda i: (0, 0)),
        ],
        out_specs=[
            pl.BlockSpec((_R, _W2P), lambda i: (i, 0)),
            pl.BlockSpec((_R, _W2P), lambda i: (i, 0)),
            pl.BlockSpec((_R, 1), lambda i: (i, 0)),
        ],
        out_shape=[
            jax.ShapeDtypeStruct((_N, _W2P), jnp.float32),
            jax.ShapeDtypeStruct((_N, _W2P), jnp.float32),
            jax.ShapeDtypeStruct((_N, 1), jnp.float32),
        ],
    )(pa, pb, z, wlT, wrT, b)


def _tcmid_body(p0_ref, p1_ref, z_ref, rinv_ref, wl_ref, wr_ref, b_ref,
                y_ref, z2_ref):
    h = (p0_ref[...] + p1_ref[...]) * rinv_ref[...] + z_ref[...]
    y_ref[...] = _dot(h, wl_ref[...])
    z2_ref[...] = _dot(h, wr_ref[...]) + b_ref[...]


def _tcmid(p0, p1, z, rinv, wlT, wrT, b, wi, wo):
    return pl.pallas_call(
        _tcmid_body,
        grid=(_GRID,),
        in_specs=[
            pl.BlockSpec((_R, wi), lambda i: (i, 0)),
            pl.BlockSpec((_R, wi), lambda i: (i, 0)),
            pl.BlockSpec((_R, wi), lambda i: (i, 0)),
            pl.BlockSpec((_R, 1), lambda i: (i, 0)),
            pl.BlockSpec((wi, wo), lambda i: (0, 0)),
            pl.BlockSpec((wi, wo), lambda i: (0, 0)),
            pl.BlockSpec((1, wo), lambda i: (0, 0)),
        ],
        out_specs=[
            pl.BlockSpec((_R, wo), lambda i: (i, 0)),
            pl.BlockSpec((_R, wo), lambda i: (i, 0)),
        ],
        out_shape=[jax.ShapeDtypeStruct((_N, wo), jnp.float32)] * 2,
    )(p0, p1, z, rinv, wlT, wrT, b)


def _tc5_body(p0_ref, p1_ref, z_ref, rinv_ref, out_ref):
    h = (p0_ref[...] + p1_ref[...]) * rinv_ref[...] + z_ref[...]
    out_ref[...] = h[:, 0:1]


def _tc5(p0, p1, z, rinv):
    return pl.pallas_call(
        _tc5_body,
        grid=(_GRID,),
        in_specs=[
            pl.BlockSpec((_R, _W4P), lambda i: (i, 0)),
            pl.BlockSpec((_R, _W4P), lambda i: (i, 0)),
            pl.BlockSpec((_R, _W4P), lambda i: (i, 0)),
            pl.BlockSpec((_R, 1), lambda i: (i, 0)),
        ],
        out_specs=pl.BlockSpec((_R, 1), lambda i: (i, 0)),
        out_shape=jax.ShapeDtypeStruct((_N, 1), jnp.float32),
    )(p0, p1, z, rinv)


def _padT(W, rows, cols):
    Wt = W.T
    return jnp.pad(Wt, ((0, rows - Wt.shape[0]), (0, cols - Wt.shape[1])))


def _padb(b, cols):
    return jnp.pad(b, (0, cols - b.shape[0])).reshape(1, cols)


def kernel(x, edge_index, W1l, b1l, W1r, W2l, b2l, W2r, W3l, b3l, W3r,
           W4l, b4l, W4r):
    f32 = jnp.float32
    src = edge_index[0]
    dst = edge_index[1]
    pad_idx = (jnp.arange(_EPAD - _E, dtype=jnp.int32) % 16)
    src_p = jnp.concatenate([src, pad_idx])
    dst_p = jnp.concatenate([dst, _N + pad_idx])
    srcr64 = src_p.reshape(_NW, 80, 64)
    dstr64 = dst_p.reshape(_NW, 80, 64)
    srcr = src_p.reshape(_NW, _NCHUNK, _LSZ)
    dstr = dst_p.reshape(_NW, _NCHUNK, _LSZ)
    srcS = jnp.stack([src_p, src_p + _N])
    srcS1 = srcS.reshape(_NC, _NS, 160, 64)
    dstS1 = dst_p.reshape(_NS, 160, 64)

    yS = _tcy1(x, _padT(W1l, _D, _W1P))
    p1 = _sc_split_factory(_W1S, 64, 160)(
        yS.reshape(2 * _N, _W1S), srcS1, dstS1, jnp.zeros((64, _W1S), f32))
    z1 = _tcz1(x, _padT(W1r, _D, _W1P), _padb(b1l, _W1P))
    y2, z2, rinv = _tc2(p1[0], p1[1], z1, _padT(W2l, _W1P, _W2P),
                        _padT(W2r, _W1P, _W2P), _padb(b2l, _W2P))
    p2 = _sc_agg_factory(_W2P, 64, 80)(y2, srcr64, dstr64,
                                       jnp.zeros((64, _W2P), f32))
    y3, z3 = _tcmid(p2[0], p2[1], z2, rinv, _padT(W3l, _W2P, _W3P),
                    _padT(W3r, _W2P, _W3P), _padb(b3l, _W3P), _W2P, _W3P)
    p3 = _sc_agg_factory(_W3P, _LSZ, _NCHUNK)(
        y3, srcr, dstr, jnp.zeros((_LSZ, _W3P), f32))
    y4, z4 = _tcmid(p3[0], p3[1], z3, rinv, _padT(W4l, _W3P, _W4P),
                    _padT(W4r, _W3P, _W4P), _padb(b4l, _W4P), _W3P, _W4P)
    p4 = _sc_agg_factory(_W4P, _LSZ, _NCHUNK)(
        y4, srcr, dstr, jnp.zeros((_LSZ, _W4P), f32))
    return _tc5(p4[0], p4[1], z4, rinv)


# speedup vs baseline: 1.1693x; 1.0960x over previous
import functools

import jax
import jax.numpy as jnp
from jax import lax
from jax.experimental import pallas as pl
from jax.experimental.pallas import tpu as pltpu
from jax.experimental.pallas import tpu_sc as plsc

_N = 10000
_E = 160000
_D = 256
_W1P, _W2P, _W3P, _W4P = 176, 128, 64, 16
_W1A, _W1B = 96, 80
_W1S = 96

_NC, _NS = 2, 16
_NW = _NC * _NS
_LSZ = 128
_NCHUNK = 40
_EPT = _NCHUNK * _LSZ
_EPAD = _NW * _EPT
_ACC_ROWS = 10112

_PREC = lax.Precision.DEFAULT


def _agg_pipeline(y_hbm, acc_sh, src_v, dst_v, bufs, gsem, ssem, nchunk):
    for b in range(3):
        pltpu.async_copy(y_hbm.at[src_v.at[b]], bufs.at[b], gsem[b])

    @pl.loop(0, nchunk, step=4)
    def _pipe(j):
        for b in range(4):
            t = j + b + 3
            bp = (b + 3) % 4

            @pl.when(t < nchunk)
            def _():
                @pl.when(t >= 4)
                def _():
                    pltpu.make_async_copy(
                        bufs.at[bp], acc_sh.at[dst_v.at[t - 4]],
                        ssem[bp]).wait()

                pltpu.async_copy(y_hbm.at[src_v.at[t]], bufs.at[bp], gsem[bp])

            pltpu.make_async_copy(y_hbm.at[src_v.at[j + b]], bufs.at[b],
                                  gsem[b]).wait()
            pltpu.async_copy(bufs.at[b], acc_sh.at[dst_v.at[j + b]], ssem[b],
                             add=True)

    for b in range(4):
        pltpu.make_async_copy(bufs.at[b],
                              acc_sh.at[dst_v.at[nchunk - 4 + b]],
                              ssem[b]).wait()


@functools.lru_cache(maxsize=None)
def _sc_agg_factory(w, lsz, nchunk):
    mesh = plsc.VectorSubcoreMesh(core_axis_name="c", subcore_axis_name="s",
                                  num_cores=_NC, num_subcores=_NS)
    rows_pt = _ACC_ROWS // _NS
    nz, zrem = divmod(rows_pt, lsz)

    @functools.partial(
        pl.kernel,
        out_type=jax.ShapeDtypeStruct((_NC, _ACC_ROWS, w), jnp.float32),
        mesh=mesh,
        scratch_types=[
            pltpu.VMEM((nchunk, lsz), jnp.int32),
            pltpu.VMEM((nchunk, lsz), jnp.int32),
            pltpu.VMEM((4, lsz, w), jnp.float32),
            pltpu.VMEM_SHARED((_ACC_ROWS, w), jnp.float32),
        ] + [pltpu.SemaphoreType.DMA] * 8,
        compiler_params=pltpu.CompilerParams(use_tc_tiling_on_sc=False),
    )
    def agg(y_hbm, src_hbm, dst_hbm, zeros_hbm, p_hbm, src_v, dst_v, bufs,
            acc_sh, *sems):
        gsem, ssem = sems[:4], sems[4:]
        c = lax.axis_index("c")
        s = lax.axis_index("s")
        wid = c * _NS + s

        pltpu.async_copy(src_hbm.at[wid], src_v, gsem[0])
        pltpu.async_copy(dst_hbm.at[wid], dst_v, gsem[0])

        pltpu.sync_copy(zeros_hbm, bufs.at[0])
        zbase = s * rows_pt
        for k in range(nz):
            pltpu.sync_copy(bufs.at[0],
                            acc_sh.at[pl.ds(zbase + k * lsz, lsz)])
        if zrem:
            pltpu.sync_copy(bufs.at[0].at[pl.ds(0, zrem)],
                            acc_sh.at[pl.ds(zbase + nz * lsz, zrem)])

        pltpu.make_async_copy(src_hbm.at[wid], src_v, gsem[0]).wait()
        pltpu.make_async_copy(dst_hbm.at[wid], dst_v, gsem[0]).wait()
        plsc.subcore_barrier()

        _agg_pipeline(y_hbm, acc_sh, src_v, dst_v, bufs, gsem, ssem, nchunk)

        plsc.subcore_barrier()
        pltpu.sync_copy(acc_sh.at[pl.ds(zbase, rows_pt)],
                        p_hbm.at[c, pl.ds(zbase, rows_pt)])

    return agg


_EPT2 = _EPAD // _NS


@functools.lru_cache(maxsize=None)
def _sc_split_factory(w, lsz, nchunk):
    mesh = plsc.VectorSubcoreMesh(core_axis_name="c", subcore_axis_name="s",
                                  num_cores=_NC, num_subcores=_NS)
    rows_pt = _ACC_ROWS // _NS
    nz, zrem = divmod(rows_pt, lsz)

    @functools.partial(
        pl.kernel,
        out_type=jax.ShapeDtypeStruct((_NC, _ACC_ROWS, w), jnp.float32),
        mesh=mesh,
        scratch_types=[
            pltpu.VMEM((nchunk, lsz), jnp.int32),
            pltpu.VMEM((nchunk, lsz), jnp.int32),
            pltpu.VMEM((4, lsz, w), jnp.float32),
            pltpu.VMEM_SHARED((_ACC_ROWS, w), jnp.float32),
        ] + [pltpu.SemaphoreType.DMA] * 8,
        compiler_params=pltpu.CompilerParams(use_tc_tiling_on_sc=False),
    )
    def agg(y_hbm, src_hbm, dst_hbm, zeros_hbm, p_hbm, src_v, dst_v, bufs,
            acc_sh, *sems):
        gsem, ssem = sems[:4], sems[4:]
        c = lax.axis_index("c")
        s = lax.axis_index("s")

        pltpu.async_copy(src_hbm.at[c, s], src_v, gsem[0])
        pltpu.async_copy(dst_hbm.at[s], dst_v, gsem[0])

        pltpu.sync_copy(zeros_hbm, bufs.at[0])
        zbase = s * rows_pt
        for k in range(nz):
            pltpu.sync_copy(bufs.at[0],
                            acc_sh.at[pl.ds(zbase + k * lsz, lsz)])
        if zrem:
            pltpu.sync_copy(bufs.at[0].at[pl.ds(0, zrem)],
                            acc_sh.at[pl.ds(zbase + nz * lsz, zrem)])

        pltpu.make_async_copy(src_hbm.at[c, s], src_v, gsem[0]).wait()
        pltpu.make_async_copy(dst_hbm.at[s], dst_v, gsem[0]).wait()
        plsc.subcore_barrier()

        _agg_pipeline(y_hbm, acc_sh, src_v, dst_v, bufs, gsem, ssem, nchunk)

        plsc.subcore_barrier()
        pltpu.sync_copy(acc_sh.at[pl.ds(zbase, rows_pt)],
                        p_hbm.at[c, pl.ds(zbase, rows_pt)])

    return agg


_R = 1000
_GRID = _N // _R


def _dot(a, b):
    return jnp.dot(a, b, preferred_element_type=jnp.float32, precision=_PREC)


def _tcy1_body(x_ref, wl_ref, ys_ref):
    y = _dot(x_ref[...], wl_ref[...])
    ya = y[:, :_W1A]
    yb = y[:, _W1A:]
    col = lax.broadcasted_iota(jnp.int32, (_R, _W1B), 1)
    yb = jnp.where(col == _W1B - 1, 1.0, yb)
    ys_ref[0] = ya
    ys_ref[1] = jnp.concatenate(
        [yb, jnp.zeros((_R, _W1S - _W1B), jnp.float32)], axis=1)


def _tcy1(x, wlT):
    return pl.pallas_call(
        _tcy1_body,
        grid=(_GRID,),
        in_specs=[
            pl.BlockSpec((_R, _D), lambda i: (i, 0)),
            pl.BlockSpec((_D, _W1P), lambda i: (0, 0)),
        ],
        out_specs=pl.BlockSpec((2, _R, _W1S), lambda i: (0, i, 0)),
        out_shape=jax.ShapeDtypeStruct((2, _N, _W1S), jnp.float32),
    )(x, wlT)


def _tcz1_body(x_ref, wr_ref, b_ref, z_ref):
    z_ref[...] = _dot(x_ref[...], wr_ref[...]) + b_ref[...]


def _tcz1(x, wrT, b):
    return pl.pallas_call(
        _tcz1_body,
        grid=(_GRID,),
        in_specs=[
            pl.BlockSpec((_R, _D), lambda i: (i, 0)),
            pl.BlockSpec((_D, _W1P), lambda i: (0, 0)),
            pl.BlockSpec((1, _W1P), lambda i: (0, 0)),
        ],
        out_specs=pl.BlockSpec((_R, _W1P), lambda i: (i, 0)),
        out_shape=jax.ShapeDtypeStruct((_N, _W1P), jnp.float32),
    )(x, wrT, b)


def _h2(pa_ref, pb_ref, z_ref):
    pa = pa_ref[...]
    pb = pb_ref[...]
    cnt = pb[:, _W1B - 1:_W1B]
    rinv = 1.0 / jnp.maximum(cnt, 1.0)
    h = jnp.concatenate([pa, pb[:, :_W1B]], axis=1) * rinv + z_ref[...]
    return h, rinv


def _tc2_body(pa_ref, pb_ref, z_ref, wl_ref, wr_ref, b_ref,
              y_ref, z2_ref, rinv_ref):
    h, rinv = _h2(pa_ref, pb_ref, z_ref)
    y_ref[...] = _dot(h, wl_ref[...])
    z2_ref[...] = _dot(h, wr_ref[...]) + b_ref[...]
    rinv_ref[...] = rinv


def _tc2(pa, pb, z, wlT, wrT, b):
    return pl.pallas_call(
        _tc2_body,
        grid=(_GRID,),
        in_specs=[
            pl.BlockSpec((_R, _W1S), lambda i: (i, 0)),
            pl.BlockSpec((_R, _W1S), lambda i: (i, 0)),
            pl.BlockSpec((_R, _W1P), lambda i: (i, 0)),
            pl.BlockSpec((_W1P, _W2P), lambda i: (0, 0)),
            pl.BlockSpec((_W1P, _W2P), lambda i: (0, 0)),
            pl.BlockSpec((1, _W2P), lambda i: (0, 0)),
        ],
        out_specs=[
            pl.BlockSpec((_R, _W2P), lambda i: (i, 0)),
            pl.BlockSpec((_R, _W2P), lambda i: (i, 0)),
            pl.BlockSpec((_R, 1), lambda i: (i, 0)),
        ],
        out_shape=[
            jax.ShapeDtypeStruct((_N, _W2P), jnp.float32),
            jax.ShapeDtypeStruct((_N, _W2P), jnp.float32),
            jax.ShapeDtypeStruct((_N, 1), jnp.float32),
        ],
    )(pa, pb, z, wlT, wrT, b)


def _tcmid_body(p0_ref, p1_ref, z_ref, rinv_ref, wl_ref, wr_ref, b_ref,
                y_ref, z2_ref):
    h = (p0_ref[...] + p1_ref[...]) * rinv_ref[...] + z_ref[...]
    y_ref[...] = _dot(h, wl_ref[...])
    z2_ref[...] = _dot(h, wr_ref[...]) + b_ref[...]


def _tcmid(p0, p1, z, rinv, wlT, wrT, b, wi, wo):
    return pl.pallas_call(
        _tcmid_body,
        grid=(_GRID,),
        in_specs=[
            pl.BlockSpec((_R, wi), lambda i: (i, 0)),
            pl.BlockSpec((_R, wi), lambda i: (i, 0)),
            pl.BlockSpec((_R, wi), lambda i: (i, 0)),
            pl.BlockSpec((_R, 1), lambda i: (i, 0)),
            pl.BlockSpec((wi, wo), lambda i: (0, 0)),
            pl.BlockSpec((wi, wo), lambda i: (0, 0)),
            pl.BlockSpec((1, wo), lambda i: (0, 0)),
        ],
        out_specs=[
            pl.BlockSpec((_R, wo), lambda i: (i, 0)),
            pl.BlockSpec((_R, wo), lambda i: (i, 0)),
        ],
        out_shape=[jax.ShapeDtypeStruct((_N, wo), jnp.float32)] * 2,
    )(p0, p1, z, rinv, wlT, wrT, b)


def _tc5_body(p0_ref, p1_ref, z_ref, rinv_ref, out_ref):
    h = (p0_ref[...] + p1_ref[...]) * rinv_ref[...] + z_ref[...]
    out_ref[...] = h[:, 0:1]


def _tc5(p0, p1, z, rinv):
    return pl.pallas_call(
        _tc5_body,
        grid=(_GRID,),
        in_specs=[
            pl.BlockSpec((_R, _W4P), lambda i: (i, 0)),
            pl.BlockSpec((_R, _W4P), lambda i: (i, 0)),
            pl.BlockSpec((_R, _W4P), lambda i: (i, 0)),
            pl.BlockSpec((_R, 1), lambda i: (i, 0)),
        ],
        out_specs=pl.BlockSpec((_R, 1), lambda i: (i, 0)),
        out_shape=jax.ShapeDtypeStruct((_N, 1), jnp.float32),
    )(p0, p1, z, rinv)


def _padT(W, rows, cols):
    Wt = W.T
    return jnp.pad(Wt, ((0, rows - Wt.shape[0]), (0, cols - Wt.shape[1])))


def _padb(b, cols):
    return jnp.pad(b, (0, cols - b.shape[0])).reshape(1, cols)


def kernel(x, edge_index, W1l, b1l, W1r, W2l, b2l, W2r, W3l, b3l, W3r,
           W4l, b4l, W4r):
    f32 = jnp.float32
    src = edge_index[0]
    dst = edge_index[1]
    pad_idx = (jnp.arange(_EPAD - _E, dtype=jnp.int32) % 16)
    src_p = jnp.concatenate([src, pad_idx])
    dst_p = jnp.concatenate([dst, _N + pad_idx])
    srcr64 = src_p.reshape(_NW, 80, 64)
    dstr64 = dst_p.reshape(_NW, 80, 64)
    srcr = src_p.reshape(_NW, _NCHUNK, _LSZ)
    dstr = dst_p.reshape(_NW, _NCHUNK, _LSZ)
    srcS = jnp.stack([src_p, src_p + _N])
    srcS1 = srcS.reshape(_NC, _NS, 160, 64)
    dstS1 = dst_p.reshape(_NS, 160, 64)

    yS = _tcy1(x, _padT(W1l, _D, _W1P))
    p1 = _sc_split_factory(_W1S, 64, 160)(
        yS.reshape(2 * _N, _W1S), srcS1, dstS1, jnp.zeros((64, _W1S), f32))
    z1 = _tcz1(x, _padT(W1r, _D, _W1P), _padb(b1l, _W1P))
    y2, z2, rinv = _tc2(p1[0], p1[1], z1, _padT(W2l, _W1P, _W2P),
                        _padT(W2r, _W1P, _W2P), _padb(b2l, _W2P))
    p2 = _sc_agg_factory(_W2P, 64, 80)(y2, srcr64, dstr64,
                                       jnp.zeros((64, _W2P), f32))
    y3, z3 = _tcmid(p2[0], p2[1], z2, rinv, _padT(W3l, _W2P, _W3P),
                    _padT(W3r, _W2P, _W3P), _padb(b3l, _W3P), _W2P, _W3P)
    p3 = _sc_agg_factory(_W3P, _LSZ, _NCHUNK)(
        y3, srcr, dstr, jnp.zeros((_LSZ, _W3P), f32))
    y4, z4 = _tcmid(p3[0], p3[1], z3, rinv, _padT(W4l, _W3P, _W4P),
                    _padT(W4r, _W3P, _W4P), _padb(b4l, _W4P), _W3P, _W4P)
    p4 = _sc_agg_factory(_W4P, _LSZ, _NCHUNK)(
        y4, srcr, dstr, jnp.zeros((_LSZ, _W4P), f32))
    return _tc5(p4[0], p4[1], z4, rinv)


# speedup vs baseline: 1.2015x; 1.0276x over previous
import functools

import jax
import jax.numpy as jnp
from jax import lax
from jax.experimental import pallas as pl
from jax.experimental.pallas import tpu as pltpu
from jax.experimental.pallas import tpu_sc as plsc

_N = 10000
_E = 160000
_D = 256
_W1P, _W2P, _W3P, _W4P = 176, 128, 64, 16
_W1A, _W1B = 96, 80
_W1S = 96

_NC, _NS = 2, 16
_NW = _NC * _NS
_LSZ = 128
_NCHUNK = 40
_EPT = _NCHUNK * _LSZ
_EPAD = _NW * _EPT
_ACC_ROWS = 10112

_PREC = lax.Precision.DEFAULT


def _agg_pipeline(y_hbm, acc_sh, src_v, dst_v, bufs, gsem, ssem, nchunk):
    for b in range(3):
        pltpu.async_copy(y_hbm.at[src_v.at[b]], bufs.at[b], gsem[b])

    @pl.loop(0, nchunk, step=4)
    def _pipe(j):
        for b in range(4):
            t = j + b + 3
            bp = (b + 3) % 4

            @pl.when(t < nchunk)
            def _():
                @pl.when(t >= 4)
                def _():
                    pltpu.make_async_copy(
                        bufs.at[bp], acc_sh.at[dst_v.at[t - 4]],
                        ssem[bp]).wait()

                pltpu.async_copy(y_hbm.at[src_v.at[t]], bufs.at[bp], gsem[bp])

            pltpu.make_async_copy(y_hbm.at[src_v.at[j + b]], bufs.at[b],
                                  gsem[b]).wait()
            pltpu.async_copy(bufs.at[b], acc_sh.at[dst_v.at[j + b]], ssem[b],
                             add=True)

    for b in range(4):
        pltpu.make_async_copy(bufs.at[b],
                              acc_sh.at[dst_v.at[nchunk - 4 + b]],
                              ssem[b]).wait()


@functools.lru_cache(maxsize=None)
def _sc_agg_factory(w, lsz, nchunk):
    mesh = plsc.VectorSubcoreMesh(core_axis_name="c", subcore_axis_name="s",
                                  num_cores=_NC, num_subcores=_NS)
    rows_pt = _ACC_ROWS // _NS
    nz, zrem = divmod(rows_pt, lsz)

    @functools.partial(
        pl.kernel,
        out_type=jax.ShapeDtypeStruct((_NC, _ACC_ROWS, w), jnp.float32),
        mesh=mesh,
        scratch_types=[
            pltpu.VMEM((nchunk, lsz), jnp.int32),
            pltpu.VMEM((nchunk, lsz), jnp.int32),
            pltpu.VMEM((4, lsz, w), jnp.float32),
            pltpu.VMEM_SHARED((_ACC_ROWS, w), jnp.float32),
        ] + [pltpu.SemaphoreType.DMA] * 8,
        compiler_params=pltpu.CompilerParams(use_tc_tiling_on_sc=False),
    )
    def agg(y_hbm, src_hbm, dst_hbm, zeros_hbm, p_hbm, src_v, dst_v, bufs,
            acc_sh, *sems):
        gsem, ssem = sems[:4], sems[4:]
        c = lax.axis_index("c")
        s = lax.axis_index("s")
        wid = c * _NS + s

        pltpu.async_copy(src_hbm.at[wid], src_v, gsem[0])
        pltpu.async_copy(dst_hbm.at[wid], dst_v, gsem[0])

        pltpu.sync_copy(zeros_hbm, bufs.at[0])
        zbase = s * rows_pt
        for k in range(nz):
            pltpu.sync_copy(bufs.at[0],
                            acc_sh.at[pl.ds(zbase + k * lsz, lsz)])
        if zrem:
            pltpu.sync_copy(bufs.at[0].at[pl.ds(0, zrem)],
                            acc_sh.at[pl.ds(zbase + nz * lsz, zrem)])

        pltpu.make_async_copy(src_hbm.at[wid], src_v, gsem[0]).wait()
        pltpu.make_async_copy(dst_hbm.at[wid], dst_v, gsem[0]).wait()
        plsc.subcore_barrier()

        _agg_pipeline(y_hbm, acc_sh, src_v, dst_v, bufs, gsem, ssem, nchunk)

        plsc.subcore_barrier()
        pltpu.sync_copy(acc_sh.at[pl.ds(zbase, rows_pt)],
                        p_hbm.at[c, pl.ds(zbase, rows_pt)])

    return agg


_EPT2 = _EPAD // _NS


@functools.lru_cache(maxsize=None)
def _sc_split_factory(w, lsz, nchunk):
    mesh = plsc.VectorSubcoreMesh(core_axis_name="c", subcore_axis_name="s",
                                  num_cores=_NC, num_subcores=_NS)
    rows_pt = _ACC_ROWS // _NS
    nz, zrem = divmod(rows_pt, lsz)

    @functools.partial(
        pl.kernel,
        out_type=jax.ShapeDtypeStruct((_NC, _ACC_ROWS, w), jnp.float32),
        mesh=mesh,
        scratch_types=[
            pltpu.VMEM((nchunk, lsz), jnp.int32),
            pltpu.VMEM((nchunk, lsz), jnp.int32),
            pltpu.VMEM((4, lsz, w), jnp.float32),
            pltpu.VMEM_SHARED((_ACC_ROWS, w), jnp.float32),
        ] + [pltpu.SemaphoreType.DMA] * 8,
        compiler_params=pltpu.CompilerParams(use_tc_tiling_on_sc=False),
    )
    def agg(y_hbm, src_hbm, dst_hbm, zeros_hbm, p_hbm, src_v, dst_v, bufs,
            acc_sh, *sems):
        gsem, ssem = sems[:4], sems[4:]
        c = lax.axis_index("c")
        s = lax.axis_index("s")

        pltpu.async_copy(src_hbm.at[c, s], src_v, gsem[0])
        pltpu.async_copy(dst_hbm.at[s], dst_v, gsem[0])

        pltpu.sync_copy(zeros_hbm, bufs.at[0])
        zbase = s * rows_pt
        for k in range(nz):
            pltpu.sync_copy(bufs.at[0],
                            acc_sh.at[pl.ds(zbase + k * lsz, lsz)])
        if zrem:
            pltpu.sync_copy(bufs.at[0].at[pl.ds(0, zrem)],
                            acc_sh.at[pl.ds(zbase + nz * lsz, zrem)])

        pltpu.make_async_copy(src_hbm.at[c, s], src_v, gsem[0]).wait()
        pltpu.make_async_copy(dst_hbm.at[s], dst_v, gsem[0]).wait()
        plsc.subcore_barrier()

        _agg_pipeline(y_hbm, acc_sh, src_v, dst_v, bufs, gsem, ssem, nchunk)

        plsc.subcore_barrier()
        pltpu.sync_copy(acc_sh.at[pl.ds(zbase, rows_pt)],
                        p_hbm.at[c, pl.ds(zbase, rows_pt)])

    return agg


_R = 2000
_GRID = _N // _R


def _dot(a, b):
    return jnp.dot(a, b, preferred_element_type=jnp.float32, precision=_PREC)


def _tcy1_body(x_ref, wl_ref, ys_ref):
    y = _dot(x_ref[...], wl_ref[...])
    ya = y[:, :_W1A]
    yb = y[:, _W1A:]
    col = lax.broadcasted_iota(jnp.int32, (_R, _W1B), 1)
    yb = jnp.where(col == _W1B - 1, 1.0, yb)
    ys_ref[0] = ya
    ys_ref[1] = jnp.concatenate(
        [yb, jnp.zeros((_R, _W1S - _W1B), jnp.float32)], axis=1)


def _tcy1(x, wlT):
    return pl.pallas_call(
        _tcy1_body,
        grid=(_GRID,),
        in_specs=[
            pl.BlockSpec((_R, _D), lambda i: (i, 0)),
            pl.BlockSpec((_D, _W1P), lambda i: (0, 0)),
        ],
        out_specs=pl.BlockSpec((2, _R, _W1S), lambda i: (0, i, 0)),
        out_shape=jax.ShapeDtypeStruct((2, _N, _W1S), jnp.float32),
    )(x, wlT)


def _tcz1_body(x_ref, wr_ref, b_ref, z_ref):
    z_ref[...] = _dot(x_ref[...], wr_ref[...]) + b_ref[...]


def _tcz1(x, wrT, b):
    return pl.pallas_call(
        _tcz1_body,
        grid=(_GRID,),
        in_specs=[
            pl.BlockSpec((_R, _D), lambda i: (i, 0)),
            pl.BlockSpec((_D, _W1P), lambda i: (0, 0)),
            pl.BlockSpec((1, _W1P), lambda i: (0, 0)),
        ],
        out_specs=pl.BlockSpec((_R, _W1P), lambda i: (i, 0)),
        out_shape=jax.ShapeDtypeStruct((_N, _W1P), jnp.float32),
    )(x, wrT, b)


def _h2(pa_ref, pb_ref, z_ref):
    pa = pa_ref[...]
    pb = pb_ref[...]
    cnt = pb[:, _W1B - 1:_W1B]
    rinv = 1.0 / jnp.maximum(cnt, 1.0)
    h = jnp.concatenate([pa, pb[:, :_W1B]], axis=1) * rinv + z_ref[...]
    return h, rinv


def _tc2_body(pa_ref, pb_ref, z_ref, wl_ref, wr_ref, b_ref,
              y_ref, z2_ref, rinv_ref):
    h, rinv = _h2(pa_ref, pb_ref, z_ref)
    y_ref[...] = _dot(h, wl_ref[...])
    z2_ref[...] = _dot(h, wr_ref[...]) + b_ref[...]
    rinv_ref[...] = rinv


def _tc2(pa, pb, z, wlT, wrT, b):
    return pl.pallas_call(
        _tc2_body,
        grid=(_GRID,),
        in_specs=[
            pl.BlockSpec((_R, _W1S), lambda i: (i, 0)),
            pl.BlockSpec((_R, _W1S), lambda i: (i, 0)),
            pl.BlockSpec((_R, _W1P), lambda i: (i, 0)),
            pl.BlockSpec((_W1P, _W2P), lambda i: (0, 0)),
            pl.BlockSpec((_W1P, _W2P), lambda i: (0, 0)),
            pl.BlockSpec((1, _W2P), lambda i: (0, 0)),
        ],
        out_specs=[
            pl.BlockSpec((_R, _W2P), lambda i: (i, 0)),
            pl.BlockSpec((_R, _W2P), lambda i: (i, 0)),
            pl.BlockSpec((_R, 1), lambda i: (i, 0)),
        ],
        out_shape=[
            jax.ShapeDtypeStruct((_N, _W2P), jnp.float32),
            jax.ShapeDtypeStruct((_N, _W2P), jnp.float32),
            jax.ShapeDtypeStruct((_N, 1), jnp.float32),
        ],
    )(pa, pb, z, wlT, wrT, b)


def _tcmid_body(p0_ref, p1_ref, z_ref, rinv_ref, wl_ref, wr_ref, b_ref,
                y_ref, z2_ref):
    h = (p0_ref[...] + p1_ref[...]) * rinv_ref[...] + z_ref[...]
    y_ref[...] = _dot(h, wl_ref[...])
    z2_ref[...] = _dot(h, wr_ref[...]) + b_ref[...]


def _tcmid(p0, p1, z, rinv, wlT, wrT, b, wi, wo):
    return pl.pallas_call(
        _tcmid_body,
        grid=(_GRID,),
        in_specs=[
            pl.BlockSpec((_R, wi), lambda i: (i, 0)),
            pl.BlockSpec((_R, wi), lambda i: (i, 0)),
            pl.BlockSpec((_R, wi), lambda i: (i, 0)),
            pl.BlockSpec((_R, 1), lambda i: (i, 0)),
            pl.BlockSpec((wi, wo), lambda i: (0, 0)),
            pl.BlockSpec((wi, wo), lambda i: (0, 0)),
            pl.BlockSpec((1, wo), lambda i: (0, 0)),
        ],
        out_specs=[
            pl.BlockSpec((_R, wo), lambda i: (i, 0)),
            pl.BlockSpec((_R, wo), lambda i: (i, 0)),
        ],
        out_shape=[jax.ShapeDtypeStruct((_N, wo), jnp.float32)] * 2,
    )(p0, p1, z, rinv, wlT, wrT, b)


def _tc5_body(p0_ref, p1_ref, z_ref, rinv_ref, out_ref):
    h = (p0_ref[...] + p1_ref[...]) * rinv_ref[...] + z_ref[...]
    out_ref[...] = h[:, 0:1]


def _tc5(p0, p1, z, rinv):
    return pl.pallas_call(
        _tc5_body,
        grid=(_GRID,),
        in_specs=[
            pl.BlockSpec((_R, _W4P), lambda i: (i, 0)),
            pl.BlockSpec((_R, _W4P), lambda i: (i, 0)),
            pl.BlockSpec((_R, _W4P), lambda i: (i, 0)),
            pl.BlockSpec((_R, 1), lambda i: (i, 0)),
        ],
        out_specs=pl.BlockSpec((_R, 1), lambda i: (i, 0)),
        out_shape=jax.ShapeDtypeStruct((_N, 1), jnp.float32),
    )(p0, p1, z, rinv)


def _padT(W, rows, cols):
    Wt = W.T
    return jnp.pad(Wt, ((0, rows - Wt.shape[0]), (0, cols - Wt.shape[1])))


def _padb(b, cols):
    return jnp.pad(b, (0, cols - b.shape[0])).reshape(1, cols)


def kernel(x, edge_index, W1l, b1l, W1r, W2l, b2l, W2r, W3l, b3l, W3r,
           W4l, b4l, W4r):
    f32 = jnp.float32
    src = edge_index[0]
    dst = edge_index[1]
    pad_idx = (jnp.arange(_EPAD - _E, dtype=jnp.int32) % 16)
    src_p = jnp.concatenate([src, pad_idx])
    dst_p = jnp.concatenate([dst, _N + pad_idx])
    srcr64 = src_p.reshape(_NW, 80, 64)
    dstr64 = dst_p.reshape(_NW, 80, 64)
    srcr = src_p.reshape(_NW, _NCHUNK, _LSZ)
    dstr = dst_p.reshape(_NW, _NCHUNK, _LSZ)
    srcS = jnp.stack([src_p, src_p + _N])
    srcS1 = srcS.reshape(_NC, _NS, 160, 64)
    dstS1 = dst_p.reshape(_NS, 160, 64)

    yS = _tcy1(x, _padT(W1l, _D, _W1P))
    p1 = _sc_split_factory(_W1S, 64, 160)(
        yS.reshape(2 * _N, _W1S), srcS1, dstS1, jnp.zeros((64, _W1S), f32))
    z1 = _tcz1(x, _padT(W1r, _D, _W1P), _padb(b1l, _W1P))
    y2, z2, rinv = _tc2(p1[0], p1[1], z1, _padT(W2l, _W1P, _W2P),
                        _padT(W2r, _W1P, _W2P), _padb(b2l, _W2P))
    p2 = _sc_agg_factory(_W2P, 64, 80)(y2, srcr64, dstr64,
                                       jnp.zeros((64, _W2P), f32))
    y3, z3 = _tcmid(p2[0], p2[1], z2, rinv, _padT(W3l, _W2P, _W3P),
                    _padT(W3r, _W2P, _W3P), _padb(b3l, _W3P), _W2P, _W3P)
    p3 = _sc_agg_factory(_W3P, _LSZ, _NCHUNK)(
        y3, srcr, dstr, jnp.zeros((_LSZ, _W3P), f32))
    y4, z4 = _tcmid(p3[0], p3[1], z3, rinv, _padT(W4l, _W3P, _W4P),
                    _padT(W4r, _W3P, _W4P), _padb(b4l, _W4P), _W3P, _W4P)
    p4 = _sc_agg_factory(_W4P, _LSZ, _NCHUNK)(
        y4, srcr, dstr, jnp.zeros((_LSZ, _W4P), f32))
    return _tc5(p4[0], p4[1], z4, rinv)
